# baseline (speedup 1.0000x reference)
import functools

import jax
import jax.numpy as jnp
from jax import lax
from jax.experimental import pallas as pl
from jax.experimental.pallas import tpu as pltpu
from jax.experimental.pallas import tpu_sc as plsc

BATCH = 16384
FIELDS = 26
DIM = 32
NROW = 1000000
B = BATCH * FIELDS

_info = plsc.get_sparse_core_info()
NC = _info.num_cores
NS = _info.num_subcores
NW = NC * NS

NTILE = NROW // 128
TAIL = NROW - NTILE * 128
GRP = 4
NG = 61
TW = GRP * NG


def _repack_body(tab_t, out_hbm, in_v, out_v0, out_v1, g_sem, s_sem):
    wid = lax.axis_index("s") * NC + lax.axis_index("c")
    rows_lo = jax.lax.iota(jnp.int32, 16)
    rows_hi = rows_lo + 16
    base = wid * TW

    def transpose_cols(buf, cols):
        ov = out_v0 if buf == 0 else out_v1
        iv = in_v.at[buf]

        @plsc.parallel_loop(0, cols, 1, unroll=8)
        def _(c):
            col = jnp.full((16,), c, jnp.int32)
            v_lo = plsc.load_gather(iv, [rows_lo, col])
            v_hi = plsc.load_gather(iv, [rows_hi, col])
            ov[pl.ds(c * 32, 16)] = v_lo
            ov[pl.ds(c * 32 + 16, 16)] = v_hi

    def _in_copy(k, buf):
        return (
            tab_t.at[:, pl.ds((base + k * GRP) * 128, GRP * 128)],
            in_v.at[buf, :, pl.ds(0, GRP * 128)],
            g_sem.at[buf],
        )

    def _out_copy(k, buf):
        return (
            out_v0 if buf == 0 else out_v1,
            out_hbm.at[pl.ds((base + k * GRP) * 4096, GRP * 4096)],
            s_sem.at[buf],
        )

    pltpu.async_copy(*_in_copy(0, 0))
    pltpu.async_copy(*_in_copy(1, 1))

    def pair_body(t2, carry):
        for b in (0, 1):
            k = t2 * 2 + b
            pltpu.make_async_copy(*_in_copy(k, b)).wait()

            @pl.when(t2 > 0)
            def _():
                pltpu.make_async_copy(*_out_copy(k - 2, b)).wait()

            transpose_cols(b, GRP * 128)
            pltpu.async_copy(*_out_copy(k, b))

            @pl.when(k + 2 < NG)
            def _():
                pltpu.async_copy(*_in_copy(k + 2, b))
        return carry

    lax.fori_loop(0, NG // 2, pair_body, 0)
    pltpu.make_async_copy(*_in_copy(NG - 1, 0)).wait()
    pltpu.make_async_copy(*_out_copy(NG - 3, 0)).wait()
    transpose_cols(0, GRP * 128)
    pltpu.async_copy(*_out_copy(NG - 1, 0))
    pltpu.make_async_copy(*_out_copy(NG - 2, 1)).wait()
    pltpu.make_async_copy(*_out_copy(NG - 1, 0)).wait()

    @pl.when(wid < NTILE - NW * TW)
    def _():
        et = NW * TW + wid
        pltpu.async_copy(
            tab_t.at[:, pl.ds(et * 128, 128)],
            in_v.at[0, :, pl.ds(0, 128)],
            g_sem.at[0],
        ).wait()
        transpose_cols(0, 128)
        pltpu.async_copy(
            out_v0.at[pl.ds(0, 4096)],
            out_hbm.at[pl.ds(et * 4096, 4096)],
            s_sem.at[0],
        ).wait()

    @pl.when(wid == 0)
    def _():
        for f in range(32):
            pltpu.async_copy(
                tab_t.at[f, pl.ds(NTILE * 128, TAIL)],
                in_v.at[1, f, pl.ds(0, TAIL)],
                g_sem.at[1],
            )
        for f in range(32):
            pltpu.make_async_copy(
                tab_t.at[f, pl.ds(NTILE * 128, TAIL)],
                in_v.at[1, f, pl.ds(0, TAIL)],
                g_sem.at[1],
            ).wait()
        transpose_cols(1, TAIL)
        pltpu.async_copy(
            out_v1.at[pl.ds(0, TAIL * 32)],
            out_hbm.at[pl.ds(NTILE * 4096, TAIL * 32)],
            s_sem.at[1],
        ).wait()


_repack = functools.partial(
    pl.kernel,
    mesh=plsc.VectorSubcoreMesh(core_axis_name="c", subcore_axis_name="s"),
    compiler_params=pltpu.CompilerParams(
        needs_layout_passes=False, use_tc_tiling_on_sc=False
    ),
    out_type=jax.ShapeDtypeStruct((NROW * DIM,), jnp.float32),
    scratch_types=[
        pltpu.VMEM((2, 32, GRP * 128 + 1), jnp.float32),
        pltpu.VMEM((GRP * 4096,), jnp.float32),
        pltpu.VMEM((GRP * 4096,), jnp.float32),
        pltpu.SemaphoreType.DMA((2,)),
        pltpu.SemaphoreType.DMA((2,)),
    ],
)(_repack_body)


CHUNK = 832
N_CHUNKS = 16
NBUF = 4
B_PER_W = CHUNK * N_CHUNKS
ROWS = B // CHUNK


def _embed_body(table_hbm, idx_hbm, out_hbm, idx_v, rows_v, g_sem, s_sem):
    wid = lax.axis_index("s") * NC + lax.axis_index("c")
    row0 = wid * N_CHUNKS
    pltpu.sync_copy(idx_hbm.at[pl.ds(row0, N_CHUNKS)], idx_v)

    hg = [None] * NBUF
    hs = [None] * NBUF
    for b in range(NBUF):
        hg[b] = pltpu.async_copy(
            table_hbm.at[idx_v.at[b]], rows_v.at[b], g_sem.at[b]
        )
    for i in range(N_CHUNKS):
        b = i % NBUF
        hg[b].wait()
        hs[b] = pltpu.async_copy(rows_v.at[b], out_hbm.at[row0 + i], s_sem.at[b])
        nxt = i + NBUF
        if nxt < N_CHUNKS:
            hs[b].wait()
            hg[b] = pltpu.async_copy(
                table_hbm.at[idx_v.at[nxt]], rows_v.at[b], g_sem.at[b]
            )
    for b in range(NBUF):
        hs[b].wait()


_embed = functools.partial(
    pl.kernel,
    mesh=plsc.VectorSubcoreMesh(core_axis_name="c", subcore_axis_name="s"),
    compiler_params=pltpu.CompilerParams(use_tc_tiling_on_sc=False),
    out_type=jax.ShapeDtypeStruct((ROWS, CHUNK, DIM), jnp.float32),
    scratch_types=[
        pltpu.VMEM((N_CHUNKS, CHUNK), jnp.int32),
        pltpu.VMEM((NBUF, CHUNK, DIM), jnp.float32),
        pltpu.SemaphoreType.DMA((NBUF,)),
        pltpu.SemaphoreType.DMA((NBUF,)),
    ],
)(_embed_body)


def kernel(inputs, embeddings):
    idx2 = inputs.reshape(ROWS, CHUNK).astype(jnp.int32)
    tab_flat = _repack(embeddings.T)
    tab = tab_flat.reshape(NROW, DIM)
    out = _embed(tab, idx2)
    return out.reshape(BATCH, FIELDS, DIM)


# speedup vs baseline: 3.7124x; 3.7124x over previous
import functools

import jax
import jax.numpy as jnp
from jax import lax
from jax.experimental import pallas as pl
from jax.experimental.pallas import tpu as pltpu
from jax.experimental.pallas import tpu_sc as plsc

BATCH = 16384
FIELDS = 26
DIM = 32
NROW = 1000000
B = BATCH * FIELDS

_info = plsc.get_sparse_core_info()
NC = _info.num_cores
NS = _info.num_subcores
NW = NC * NS

NTILE = NROW // 128
TAIL = NROW - NTILE * 128
GRP = 4
NG = 61
TW = GRP * NG


def _repack_body(tab_t, out_hbm, in_v, out_v0, out_v1, g_sem, s_sem):
    wid = lax.axis_index("s") * NC + lax.axis_index("c")
    rows_lo = jax.lax.iota(jnp.int32, 16)
    rows_hi = rows_lo + 16
    base = wid * TW

    def transpose_cols(buf, cols):
        ov = out_v0 if buf == 0 else out_v1
        iv = in_v.at[buf]

        @plsc.parallel_loop(0, cols, 1, unroll=8)
        def _(c):
            col = jnp.full((16,), c, jnp.int32)
            v_lo = plsc.load_gather(iv, [rows_lo, col])
            v_hi = plsc.load_gather(iv, [rows_hi, col])
            ov[pl.ds(c * 32, 16)] = v_lo
            ov[pl.ds(c * 32 + 16, 16)] = v_hi

    def _in_copy(k, buf):
        return (
            tab_t.at[:, pl.ds((base + k * GRP) * 128, GRP * 128)],
            in_v.at[buf, :, pl.ds(0, GRP * 128)],
            g_sem.at[buf],
        )

    def _out_copy(k, buf):
        return (
            out_v0 if buf == 0 else out_v1,
            out_hbm.at[pl.ds((base + k * GRP) * 4096, GRP * 4096)],
            s_sem.at[buf],
        )

    pltpu.async_copy(*_in_copy(0, 0))
    pltpu.async_copy(*_in_copy(1, 1))

    def pair_body(t2, carry):
        for b in (0, 1):
            k = t2 * 2 + b
            pltpu.make_async_copy(*_in_copy(k, b)).wait()

            @pl.when(t2 > 0)
            def _():
                pltpu.make_async_copy(*_out_copy(k - 2, b)).wait()

            transpose_cols(b, GRP * 128)
            pltpu.async_copy(*_out_copy(k, b))

            @pl.when(k + 2 < NG)
            def _():
                pltpu.async_copy(*_in_copy(k + 2, b))
        return carry

    lax.fori_loop(0, NG // 2, pair_body, 0)
    pltpu.make_async_copy(*_in_copy(NG - 1, 0)).wait()
    pltpu.make_async_copy(*_out_copy(NG - 3, 0)).wait()
    transpose_cols(0, GRP * 128)
    pltpu.async_copy(*_out_copy(NG - 1, 0))
    pltpu.make_async_copy(*_out_copy(NG - 2, 1)).wait()
    pltpu.make_async_copy(*_out_copy(NG - 1, 0)).wait()

    @pl.when(wid < NTILE - NW * TW)
    def _():
        et = NW * TW + wid
        pltpu.async_copy(
            tab_t.at[:, pl.ds(et * 128, 128)],
            in_v.at[0, :, pl.ds(0, 128)],
            g_sem.at[0],
        ).wait()
        transpose_cols(0, 128)
        pltpu.async_copy(
            out_v0.at[pl.ds(0, 4096)],
            out_hbm.at[pl.ds(et * 4096, 4096)],
            s_sem.at[0],
        ).wait()

    @pl.when(wid == 0)
    def _():
        for f in range(32):
            pltpu.async_copy(
                tab_t.at[f, pl.ds(NTILE * 128, TAIL)],
                in_v.at[1, f, pl.ds(0, TAIL)],
                g_sem.at[1],
            )
        for f in range(32):
            pltpu.make_async_copy(
                tab_t.at[f, pl.ds(NTILE * 128, TAIL)],
                in_v.at[1, f, pl.ds(0, TAIL)],
                g_sem.at[1],
            ).wait()
        transpose_cols(1, TAIL)
        pltpu.async_copy(
            out_v1.at[pl.ds(0, TAIL * 32)],
            out_hbm.at[pl.ds(NTILE * 4096, TAIL * 32)],
            s_sem.at[1],
        ).wait()


_repack = functools.partial(
    pl.kernel,
    mesh=plsc.VectorSubcoreMesh(core_axis_name="c", subcore_axis_name="s"),
    compiler_params=pltpu.CompilerParams(needs_layout_passes=False),
    out_type=jax.ShapeDtypeStruct((NROW * DIM,), jnp.float32),
    scratch_types=[
        pltpu.VMEM((2, 32, GRP * 128 + 1), jnp.float32),
        pltpu.VMEM((GRP * 4096,), jnp.float32),
        pltpu.VMEM((GRP * 4096,), jnp.float32),
        pltpu.SemaphoreType.DMA((2,)),
        pltpu.SemaphoreType.DMA((2,)),
    ],
)(_repack_body)


CHUNK = 832
N_CHUNKS = 16
NBUF = 4
B_PER_W = CHUNK * N_CHUNKS
ROWS = B // CHUNK


def _embed_body(table_hbm, idx_hbm, out_hbm, idx_v, rows_v, g_sem, s_sem):
    wid = lax.axis_index("s") * NC + lax.axis_index("c")
    row0 = wid * N_CHUNKS
    pltpu.sync_copy(idx_hbm.at[pl.ds(row0, N_CHUNKS)], idx_v)

    hg = [None] * NBUF
    hs = [None] * NBUF
    for b in range(NBUF):
        hg[b] = pltpu.async_copy(
            table_hbm.at[idx_v.at[b]], rows_v.at[b], g_sem.at[b]
        )
    for i in range(N_CHUNKS):
        b = i % NBUF
        hg[b].wait()
        hs[b] = pltpu.async_copy(rows_v.at[b], out_hbm.at[row0 + i], s_sem.at[b])
        nxt = i + NBUF
        if nxt < N_CHUNKS:
            hs[b].wait()
            hg[b] = pltpu.async_copy(
                table_hbm.at[idx_v.at[nxt]], rows_v.at[b], g_sem.at[b]
            )
    for b in range(NBUF):
        hs[b].wait()


_embed = functools.partial(
    pl.kernel,
    mesh=plsc.VectorSubcoreMesh(core_axis_name="c", subcore_axis_name="s"),
    compiler_params=pltpu.CompilerParams(use_tc_tiling_on_sc=False),
    out_type=jax.ShapeDtypeStruct((ROWS, CHUNK, DIM), jnp.float32),
    scratch_types=[
        pltpu.VMEM((N_CHUNKS, CHUNK), jnp.int32),
        pltpu.VMEM((NBUF, CHUNK, DIM), jnp.float32),
        pltpu.SemaphoreType.DMA((NBUF,)),
        pltpu.SemaphoreType.DMA((NBUF,)),
    ],
)(_embed_body)


def kernel(inputs, embeddings):
    idx2 = inputs.reshape(ROWS, CHUNK).astype(jnp.int32)
    out = _embed(embeddings, idx2)
    return out.reshape(BATCH, FIELDS, DIM)


# speedup vs baseline: 3.7140x; 1.0004x over previous
import functools

import jax
import jax.numpy as jnp
from jax import lax
from jax.experimental import pallas as pl
from jax.experimental.pallas import tpu as pltpu
from jax.experimental.pallas import tpu_sc as plsc

BATCH = 16384
FIELDS = 26
DIM = 32
B = BATCH * FIELDS

_info = plsc.get_sparse_core_info()
NC = _info.num_cores
NS = _info.num_subcores
NW = NC * NS

CHUNK = 832
N_CHUNKS = 16
NBUF = 4
B_PER_W = CHUNK * N_CHUNKS
ROWS = B // CHUNK


def _embed_body(table_hbm, idx_hbm, out_hbm, idx_v, rows_v, g_sem, s_sem):
    wid = lax.axis_index("s") * NC + lax.axis_index("c")
    row0 = wid * N_CHUNKS
    pltpu.sync_copy(idx_hbm.at[pl.ds(row0, N_CHUNKS)], idx_v)

    hg = [None] * NBUF
    hs = [None] * NBUF
    for b in range(NBUF):
        hg[b] = pltpu.async_copy(
            table_hbm.at[idx_v.at[b]], rows_v.at[b], g_sem.at[b]
        )
    for i in range(N_CHUNKS):
        b = i % NBUF
        hg[b].wait()
        hs[b] = pltpu.async_copy(rows_v.at[b], out_hbm.at[row0 + i], s_sem.at[b])
        nxt = i + NBUF
        if nxt < N_CHUNKS:
            hs[b].wait()
            hg[b] = pltpu.async_copy(
                table_hbm.at[idx_v.at[nxt]], rows_v.at[b], g_sem.at[b]
            )
    for b in range(NBUF):
        hs[b].wait()


_embed = functools.partial(
    pl.kernel,
    mesh=plsc.VectorSubcoreMesh(core_axis_name="c", subcore_axis_name="s"),
    compiler_params=pltpu.CompilerParams(use_tc_tiling_on_sc=False),
    out_type=jax.ShapeDtypeStruct((ROWS, CHUNK, DIM), jnp.float32),
    scratch_types=[
        pltpu.VMEM((N_CHUNKS, CHUNK), jnp.int32),
        pltpu.VMEM((NBUF, CHUNK, DIM), jnp.float32),
        pltpu.SemaphoreType.DMA((NBUF,)),
        pltpu.SemaphoreType.DMA((NBUF,)),
    ],
)(_embed_body)


def kernel(inputs, embeddings):
    idx2 = inputs.reshape(ROWS, CHUNK).astype(jnp.int32)
    out = _embed(embeddings, idx2)
    return out.reshape(BATCH, FIELDS, DIM)
